# baseline (speedup 1.0000x reference)
import functools

import jax
import jax.numpy as jnp
from jax import lax
from jax.experimental import pallas as pl
from jax.experimental.pallas import tpu as pltpu
from jax.experimental.pallas import tpu_sc as plsc

N = 10000
E = 320000
D = 128
NL = 3

NC = 2
NS = 16
LANES = 16
NW = NC * NS
CE = E // NW
KCH = 80
NCH = CE // KCH

DROWS = 752
NACC = N + DROWS
ZSUB = NACC // NS
NZSUB = 10
RSUB = N // NZSUB

BE = 2560
GE = E // BE
BNODE = 2000
GNODE = N // BNODE

_mesh = plsc.VectorSubcoreMesh(
    core_axis_name="c", subcore_axis_name="s", num_cores=NC, num_subcores=NS)

_f32 = jnp.float32


def _sc_gather_sum_body(p_hbm, q_hbm, src_hbm, dst_hbm, g_hbm,
                        sslab, dslab, buf0, buf1, sem):
    wid = lax.axis_index("s") * NC + lax.axis_index("c")
    base = wid * CE

    pltpu.sync_copy(src_hbm.at[pl.ds(base, CE)], sslab)
    pltpu.sync_copy(dst_hbm.at[pl.ds(base, CE)], dslab)
    bufs = (buf0, buf1)

    def drain(buf):
        pltpu.make_async_copy(g_hbm.at[pl.ds(base, KCH)], buf, sem).wait()

    def stage(i, buf, dr):
        co = i * KCH
        if dr:
            drain(buf)
        pltpu.sync_copy(p_hbm.at[sslab.at[pl.ds(co, KCH)]], buf)
        pltpu.sync_copy(q_hbm.at[dslab.at[pl.ds(co, KCH)]], buf, add=True)
        pltpu.async_copy(buf, g_hbm.at[pl.ds(base + co, KCH)], sem)

    stage(0, buf0, False)
    stage(1, buf1, False)

    def chunk2(j, carry):
        for b in range(2):
            stage(j * 2 + b, bufs[b], True)
        return carry

    lax.fori_loop(1, NCH // 2, chunk2, 0, unroll=False)
    stage(NCH - 1, buf0, True)
    drain(buf0)
    drain(buf1)


_sc_gather_sum = functools.partial(
    pl.kernel,
    out_type=jax.ShapeDtypeStruct((E, D), _f32),
    mesh=_mesh,
    scratch_types=[
        pltpu.VMEM((CE,), jnp.int32),
        pltpu.VMEM((CE,), jnp.int32),
        pltpu.VMEM((KCH, D), _f32),
        pltpu.VMEM((KCH, D), _f32),
        pltpu.SemaphoreType.DMA,
    ],
)(_sc_gather_sum_body)


def _sc_aggregate_body(v_hbm, src_hbm, dst_hbm, logit_hbm, gmax_hbm,
                       num_out, den_out,
                       sslab, lbuf, didx, didxp, didx8, exbuf,
                       rowbuf, exrow, gbuf, acc_sh):
    cid = lax.axis_index("c")
    sid = lax.axis_index("s")
    wid = sid * NC + cid

    pltpu.sync_copy(gmax_hbm, gbuf)
    base = wid * CE
    pltpu.sync_copy(src_hbm.at[pl.ds(base, CE)], sslab)
    gv = gbuf[...]

    zero = jnp.zeros((LANES,), _f32)

    def zrow2(k, c2):
        for j in range(D // LANES):
            exrow[k, pl.ds(j * LANES, LANES)] = zero
        return c2

    lax.fori_loop(0, KCH, zrow2, 0, unroll=False)

    for t in range(ZSUB // KCH):
        pltpu.sync_copy(exrow, acc_sh.at[pl.ds(sid * ZSUB + t * KCH, KCH)])
    pltpu.sync_copy(exrow.at[pl.ds(0, ZSUB % KCH)],
                    acc_sh.at[pl.ds(sid * ZSUB + ZSUB - ZSUB % KCH,
                                    ZSUB % KCH)])
    plsc.subcore_barrier()

    def chunk(i, carry):
        co = i * KCH
        pltpu.sync_copy(logit_hbm.at[pl.ds(base + co, KCH)], lbuf)
        pltpu.sync_copy(dst_hbm.at[pl.ds(base + co, KCH)], didx)
        lane0 = lax.iota(jnp.int32, LANES)
        for g in range(KCH // LANES):
            sl = pl.ds(g * LANES, LANES)
            dv = didx[sl]
            didxp[sl] = dv
            didx8[sl] = (dv >> 4) + N
            exbuf[sl] = jnp.exp(lbuf[sl] - gv)
        pltpu.sync_copy(v_hbm.at[sslab.at[pl.ds(co, KCH)]], rowbuf)

        def row(k, c2):
            s = exbuf[pl.ds(k, LANES)][0]
            dk = didxp[pl.ds(k, LANES)][0]
            slot = dk & 15
            lane = (slot >> 1) * LANES
            pos = (slot & 1) * 8
            for j in range(D // LANES):
                sl2 = pl.ds(j * LANES, LANES)
                rowbuf[k, sl2] = rowbuf[k, sl2] * s
                exrow[k, sl2] = zero
            sv = jnp.where(lane0 == pos, s, 0.0)
            exrow[k, pl.ds(lane, LANES)] = sv
            return c2

        lax.fori_loop(0, KCH, row, 0, unroll=False)
        pltpu.sync_copy(rowbuf, acc_sh.at[didx], add=True)
        pltpu.sync_copy(exrow, acc_sh.at[didx8], add=True)
        return carry

    lax.fori_loop(0, NCH, chunk, 0, unroll=False)

    plsc.subcore_barrier()

    @pl.when(sid < NZSUB)
    def _():
        pltpu.sync_copy(acc_sh.at[pl.ds(sid * RSUB, RSUB)],
                        num_out.at[cid, pl.ds(sid * RSUB, RSUB)])

    @pl.when(sid == NZSUB)
    def _():
        pltpu.sync_copy(acc_sh.at[pl.ds(N, DROWS)], den_out.at[cid])


_sc_aggregate = functools.partial(
    pl.kernel,
    out_type=(
        jax.ShapeDtypeStruct((NC, N, D), _f32),
        jax.ShapeDtypeStruct((NC, DROWS, D), _f32),
    ),
    mesh=_mesh,
    scratch_types=[
        pltpu.VMEM((CE,), jnp.int32),
        pltpu.VMEM((KCH,), _f32),
        pltpu.VMEM((KCH,), jnp.int32),
        pltpu.VMEM((KCH + LANES,), jnp.int32),
        pltpu.VMEM((KCH,), jnp.int32),
        pltpu.VMEM((KCH + LANES,), _f32),
        pltpu.VMEM((KCH, D), _f32),
        pltpu.VMEM((KCH, D), _f32),
        pltpu.VMEM((LANES,), _f32),
        pltpu.VMEM_SHARED((NACC, D), _f32),
    ],
)(_sc_aggregate_body)


def _pqv_body(h_ref, ws_ref, wd_ref, wv_ref, p_ref, q_ref, v_ref):
    hb = h_ref[...]
    p_ref[...] = jnp.dot(hb, ws_ref[...], preferred_element_type=_f32)
    q_ref[...] = jnp.dot(hb, wd_ref[...], preferred_element_type=_f32)
    v_ref[...] = jnp.dot(hb, wv_ref[...], preferred_element_type=_f32)


_pqv = pl.pallas_call(
    _pqv_body,
    grid=(GNODE,),
    in_specs=[
        pl.BlockSpec((BNODE, D), lambda i: (i, 0)),
        pl.BlockSpec((D, D), lambda i: (0, 0)),
        pl.BlockSpec((D, D), lambda i: (0, 0)),
        pl.BlockSpec((D, D), lambda i: (0, 0)),
    ],
    out_specs=[
        pl.BlockSpec((BNODE, D), lambda i: (i, 0)),
        pl.BlockSpec((BNODE, D), lambda i: (i, 0)),
        pl.BlockSpec((BNODE, D), lambda i: (i, 0)),
    ],
    out_shape=[jax.ShapeDtypeStruct((N, D), _f32)] * 3,
)


def _edge_body(e_ref, g_ref, we_ref, attn_ref,
               ehat_ref, logit_ref, stats_ref, lmax_ref):
    i = pl.program_id(0)
    ehat = g_ref[...] + jnp.dot(e_ref[...], we_ref[...],
                                preferred_element_type=_f32)
    ehat_ref[...] = ehat
    lr = jnp.where(ehat > 0, ehat, 0.2 * ehat)
    logit_row = lax.dot_general(attn_ref[...], lr, (((1,), (1,)), ((), ())),
                                preferred_element_type=_f32)
    logit_ref[...] = logit_row.reshape(1, 1, BE)

    @pl.when(i == 0)
    def _():
        stats_ref[...] = jnp.zeros_like(stats_ref)
        lmax_ref[...] = jnp.full_like(lmax_ref, -jnp.inf)

    stats_ref[0:1, :] += jnp.sum(ehat, axis=0, keepdims=True)
    stats_ref[1:2, :] += jnp.sum(ehat * ehat, axis=0, keepdims=True)
    lmax_ref[...] = jnp.maximum(lmax_ref[...], jnp.max(logit_row))


_edge_stage = pl.pallas_call(
    _edge_body,
    grid=(GE,),
    in_specs=[
        pl.BlockSpec((BE, D), lambda i: (i, 0)),
        pl.BlockSpec((BE, D), lambda i: (i, 0)),
        pl.BlockSpec((D, D), lambda i: (0, 0)),
        pl.BlockSpec((1, D), lambda i: (0, 0)),
    ],
    out_specs=[
        pl.BlockSpec((BE, D), lambda i: (i, 0)),
        pl.BlockSpec((1, 1, BE), lambda i: (i, 0, 0)),
        pl.BlockSpec((8, 128), lambda i: (0, 0)),
        pl.BlockSpec((8, 128), lambda i: (0, 0)),
    ],
    out_shape=[
        jax.ShapeDtypeStruct((E, D), _f32),
        jax.ShapeDtypeStruct((GE, 1, BE), _f32),
        jax.ShapeDtypeStruct((8, 128), _f32),
        jax.ShapeDtypeStruct((8, 128), _f32),
    ],
)


def _node_body(num_ref, den_ref, h_ref, gam_ref, bet_ref, out_ref):
    num = num_ref[0] + num_ref[1]
    den = den_ref[0, :N, :] + den_ref[1, :N, :]
    agg = num / (den[:, 0:1] + 1e-16)
    mu = jnp.mean(agg, axis=0, keepdims=True)
    var = jnp.mean(agg * agg, axis=0, keepdims=True) - mu * mu
    y = gam_ref[...] * (agg - mu) / jnp.sqrt(var + 1e-5) + bet_ref[...]
    out_ref[...] = jnp.maximum(y, 0.0) + h_ref[...]


_node_update = pl.pallas_call(
    _node_body,
    grid=(1,),
    in_specs=[
        pl.BlockSpec((NC, N, D), lambda i: (0, 0, 0)),
        pl.BlockSpec((NC, DROWS * 16, 8), lambda i: (0, 0, 0)),
        pl.BlockSpec((N, D), lambda i: (0, 0)),
        pl.BlockSpec((1, D), lambda i: (0, 0)),
        pl.BlockSpec((1, D), lambda i: (0, 0)),
    ],
    out_specs=pl.BlockSpec((N, D), lambda i: (0, 0)),
    out_shape=jax.ShapeDtypeStruct((N, D), _f32),
)


def _eupd_body(ehat_ref, e_ref, stats_ref, gam_ref, bet_ref, out_ref):
    s1 = stats_ref[0:1, :]
    s2 = stats_ref[1:2, :]
    mu = s1 * (1.0 / E)
    var = s2 * (1.0 / E) - mu * mu
    ehat = ehat_ref[...]
    y = gam_ref[...] * (ehat - mu) / jnp.sqrt(var + 1e-5) + bet_ref[...]
    out_ref[...] = jnp.maximum(y, 0.0) + e_ref[...]


_edge_update = pl.pallas_call(
    _eupd_body,
    grid=(GE,),
    in_specs=[
        pl.BlockSpec((BE, D), lambda i: (i, 0)),
        pl.BlockSpec((BE, D), lambda i: (i, 0)),
        pl.BlockSpec((8, 128), lambda i: (0, 0)),
        pl.BlockSpec((1, D), lambda i: (0, 0)),
        pl.BlockSpec((1, D), lambda i: (0, 0)),
    ],
    out_specs=pl.BlockSpec((BE, D), lambda i: (i, 0)),
    out_shape=jax.ShapeDtypeStruct((E, D), _f32),
)


def kernel(h, e, edge_index, Wsrc, Wdst, We, Wv, attn,
           gamma_h, beta_h, gamma_e, beta_e):
    src = edge_index[0].astype(jnp.int32)
    dst = edge_index[1].astype(jnp.int32)
    for l in range(NL):
        p, q, v = _pqv(h, Wsrc[l], Wdst[l], Wv[l])
        g = _sc_gather_sum(p, q, src, dst)
        ehat, logit3, stats, lmax = _edge_stage(
            e, g, We[l], attn[l].reshape(1, D))
        gvec = jnp.full((LANES,), jnp.max(lmax), _f32)
        num, denp = _sc_aggregate(v, src, dst, logit3.reshape(E), gvec)
        den16 = denp.reshape(NC, DROWS * 16, 8)
        h = _node_update(num, den16, h,
                         gamma_h[l].reshape(1, D), beta_h[l].reshape(1, D))
        e = _edge_update(ehat, e, stats,
                         gamma_e[l].reshape(1, D), beta_e[l].reshape(1, D))
    return (h, e)


# speedup vs baseline: 1.0513x; 1.0513x over previous
import functools

import jax
import jax.numpy as jnp
from jax import lax
from jax.experimental import pallas as pl
from jax.experimental.pallas import tpu as pltpu
from jax.experimental.pallas import tpu_sc as plsc

N = 10000
E = 320000
D = 128
NL = 3

NC = 2
NS = 16
LANES = 16
NW = NC * NS
CE = E // NW
KCH = 80
NCH = CE // KCH

DROWS = 752
NACC = N + DROWS
ZSUB = NACC // NS
NZSUB = 10
RSUB = N // NZSUB

BE = 2560
GE = E // BE
BNODE = 2000
GNODE = N // BNODE

_mesh = plsc.VectorSubcoreMesh(
    core_axis_name="c", subcore_axis_name="s", num_cores=NC, num_subcores=NS)

_f32 = jnp.float32


def _sc_gather_sum_body(p_hbm, q_hbm, src_hbm, dst_hbm, g_hbm,
                        sslab, dslab, buf0, buf1, semp, sems):
    wid = lax.axis_index("s") * NC + lax.axis_index("c")
    base = wid * CE

    pltpu.sync_copy(src_hbm.at[pl.ds(base, CE)], sslab)
    pltpu.sync_copy(dst_hbm.at[pl.ds(base, CE)], dslab)

    def wait_one(buf, sm):
        pltpu.make_async_copy(g_hbm.at[pl.ds(base, KCH)], buf, sm).wait()

    def stage(i, b, nb, dr, pf):
        co = i * KCH
        wait_one(b, semp)
        if pf:
            if dr:
                wait_one(nb, sems)
            pltpu.async_copy(p_hbm.at[sslab.at[pl.ds(co + KCH, KCH)]],
                             nb, semp)
        pltpu.sync_copy(q_hbm.at[dslab.at[pl.ds(co, KCH)]], b, add=True)
        pltpu.async_copy(b, g_hbm.at[pl.ds(base + co, KCH)], sems)

    pltpu.async_copy(p_hbm.at[sslab.at[pl.ds(0, KCH)]], buf0, semp)
    stage(0, buf0, buf1, False, True)

    def chunk2(j, carry):
        i = j * 2 + 1
        stage(i, buf1, buf0, True, True)
        stage(i + 1, buf0, buf1, True, True)
        return carry

    lax.fori_loop(0, (NCH - 3) // 2, chunk2, 0, unroll=False)
    stage(NCH - 2, buf1, buf0, True, True)
    stage(NCH - 1, buf0, buf1, False, False)
    wait_one(buf0, sems)
    wait_one(buf1, sems)


_sc_gather_sum = functools.partial(
    pl.kernel,
    out_type=jax.ShapeDtypeStruct((E, D), _f32),
    mesh=_mesh,
    scratch_types=[
        pltpu.VMEM((CE,), jnp.int32),
        pltpu.VMEM((CE,), jnp.int32),
        pltpu.VMEM((KCH, D), _f32),
        pltpu.VMEM((KCH, D), _f32),
        pltpu.SemaphoreType.DMA,
        pltpu.SemaphoreType.DMA,
    ],
)(_sc_gather_sum_body)


def _sc_aggregate_body(v_hbm, src_hbm, dst_hbm, logit_hbm, gmax_hbm,
                       num_out, den_out,
                       sslab, lbuf, didx, didxp, didx8, exbuf,
                       rowbuf, exrow, gbuf, acc_sh):
    cid = lax.axis_index("c")
    sid = lax.axis_index("s")
    wid = sid * NC + cid

    pltpu.sync_copy(gmax_hbm, gbuf)
    base = wid * CE
    pltpu.sync_copy(src_hbm.at[pl.ds(base, CE)], sslab)
    gv = gbuf[...]

    zero = jnp.zeros((LANES,), _f32)

    def zrow2(k, c2):
        for j in range(D // LANES):
            exrow[k, pl.ds(j * LANES, LANES)] = zero
        return c2

    lax.fori_loop(0, KCH, zrow2, 0, unroll=False)

    for t in range(ZSUB // KCH):
        pltpu.sync_copy(exrow, acc_sh.at[pl.ds(sid * ZSUB + t * KCH, KCH)])
    pltpu.sync_copy(exrow.at[pl.ds(0, ZSUB % KCH)],
                    acc_sh.at[pl.ds(sid * ZSUB + ZSUB - ZSUB % KCH,
                                    ZSUB % KCH)])
    plsc.subcore_barrier()

    def chunk(i, carry):
        co = i * KCH
        pltpu.sync_copy(logit_hbm.at[pl.ds(base + co, KCH)], lbuf)
        pltpu.sync_copy(dst_hbm.at[pl.ds(base + co, KCH)], didx)
        lane0 = lax.iota(jnp.int32, LANES)
        for g in range(KCH // LANES):
            sl = pl.ds(g * LANES, LANES)
            dv = didx[sl]
            didxp[sl] = dv
            didx8[sl] = (dv >> 4) + N
            exbuf[sl] = jnp.exp(lbuf[sl] - gv)
        pltpu.sync_copy(v_hbm.at[sslab.at[pl.ds(co, KCH)]], rowbuf)

        def row(k, c2):
            s = exbuf[pl.ds(k, LANES)][0]
            dk = didxp[pl.ds(k, LANES)][0]
            slot = dk & 15
            lane = (slot >> 1) * LANES
            pos = (slot & 1) * 8
            for j in range(D // LANES):
                sl2 = pl.ds(j * LANES, LANES)
                rowbuf[k, sl2] = rowbuf[k, sl2] * s
                exrow[k, sl2] = zero
            sv = jnp.where(lane0 == pos, s, 0.0)
            exrow[k, pl.ds(lane, LANES)] = sv
            return c2

        lax.fori_loop(0, KCH, row, 0, unroll=False)
        pltpu.sync_copy(rowbuf, acc_sh.at[didx], add=True)
        pltpu.sync_copy(exrow, acc_sh.at[didx8], add=True)
        return carry

    lax.fori_loop(0, NCH, chunk, 0, unroll=False)

    plsc.subcore_barrier()

    @pl.when(sid < NZSUB)
    def _():
        pltpu.sync_copy(acc_sh.at[pl.ds(sid * RSUB, RSUB)],
                        num_out.at[cid, pl.ds(sid * RSUB, RSUB)])

    @pl.when(sid == NZSUB)
    def _():
        pltpu.sync_copy(acc_sh.at[pl.ds(N, DROWS)], den_out.at[cid])


_sc_aggregate = functools.partial(
    pl.kernel,
    out_type=(
        jax.ShapeDtypeStruct((NC, N, D), _f32),
        jax.ShapeDtypeStruct((NC, DROWS, D), _f32),
    ),
    mesh=_mesh,
    scratch_types=[
        pltpu.VMEM((CE,), jnp.int32),
        pltpu.VMEM((KCH,), _f32),
        pltpu.VMEM((KCH,), jnp.int32),
        pltpu.VMEM((KCH + LANES,), jnp.int32),
        pltpu.VMEM((KCH,), jnp.int32),
        pltpu.VMEM((KCH + LANES,), _f32),
        pltpu.VMEM((KCH, D), _f32),
        pltpu.VMEM((KCH, D), _f32),
        pltpu.VMEM((LANES,), _f32),
        pltpu.VMEM_SHARED((NACC, D), _f32),
    ],
)(_sc_aggregate_body)


def _pqv_body(h_ref, ws_ref, wd_ref, wv_ref, p_ref, q_ref, v_ref):
    hb = h_ref[...]
    p_ref[...] = jnp.dot(hb, ws_ref[...], preferred_element_type=_f32)
    q_ref[...] = jnp.dot(hb, wd_ref[...], preferred_element_type=_f32)
    v_ref[...] = jnp.dot(hb, wv_ref[...], preferred_element_type=_f32)


_pqv = pl.pallas_call(
    _pqv_body,
    grid=(GNODE,),
    in_specs=[
        pl.BlockSpec((BNODE, D), lambda i: (i, 0)),
        pl.BlockSpec((D, D), lambda i: (0, 0)),
        pl.BlockSpec((D, D), lambda i: (0, 0)),
        pl.BlockSpec((D, D), lambda i: (0, 0)),
    ],
    out_specs=[
        pl.BlockSpec((BNODE, D), lambda i: (i, 0)),
        pl.BlockSpec((BNODE, D), lambda i: (i, 0)),
        pl.BlockSpec((BNODE, D), lambda i: (i, 0)),
    ],
    out_shape=[jax.ShapeDtypeStruct((N, D), _f32)] * 3,
)


def _edge_body(e_ref, g_ref, we_ref, attn_ref,
               ehat_ref, logit_ref, stats_ref, lmax_ref):
    i = pl.program_id(0)
    ehat = g_ref[...] + jnp.dot(e_ref[...], we_ref[...],
                                preferred_element_type=_f32)
    ehat_ref[...] = ehat
    lr = jnp.where(ehat > 0, ehat, 0.2 * ehat)
    logit_row = lax.dot_general(attn_ref[...], lr, (((1,), (1,)), ((), ())),
                                preferred_element_type=_f32)
    logit_ref[...] = logit_row.reshape(1, 1, BE)

    @pl.when(i == 0)
    def _():
        stats_ref[...] = jnp.zeros_like(stats_ref)
        lmax_ref[...] = jnp.full_like(lmax_ref, -jnp.inf)

    stats_ref[0:1, :] += jnp.sum(ehat, axis=0, keepdims=True)
    stats_ref[1:2, :] += jnp.sum(ehat * ehat, axis=0, keepdims=True)
    lmax_ref[...] = jnp.maximum(lmax_ref[...], jnp.max(logit_row))


_edge_stage = pl.pallas_call(
    _edge_body,
    grid=(GE,),
    in_specs=[
        pl.BlockSpec((BE, D), lambda i: (i, 0)),
        pl.BlockSpec((BE, D), lambda i: (i, 0)),
        pl.BlockSpec((D, D), lambda i: (0, 0)),
        pl.BlockSpec((1, D), lambda i: (0, 0)),
    ],
    out_specs=[
        pl.BlockSpec((BE, D), lambda i: (i, 0)),
        pl.BlockSpec((1, 1, BE), lambda i: (i, 0, 0)),
        pl.BlockSpec((8, 128), lambda i: (0, 0)),
        pl.BlockSpec((8, 128), lambda i: (0, 0)),
    ],
    out_shape=[
        jax.ShapeDtypeStruct((E, D), _f32),
        jax.ShapeDtypeStruct((GE, 1, BE), _f32),
        jax.ShapeDtypeStruct((8, 128), _f32),
        jax.ShapeDtypeStruct((8, 128), _f32),
    ],
)


def _node_body(num_ref, den_ref, h_ref, gam_ref, bet_ref, out_ref):
    num = num_ref[0] + num_ref[1]
    den = den_ref[0, :N, :] + den_ref[1, :N, :]
    agg = num / (den[:, 0:1] + 1e-16)
    mu = jnp.mean(agg, axis=0, keepdims=True)
    var = jnp.mean(agg * agg, axis=0, keepdims=True) - mu * mu
    y = gam_ref[...] * (agg - mu) / jnp.sqrt(var + 1e-5) + bet_ref[...]
    out_ref[...] = jnp.maximum(y, 0.0) + h_ref[...]


_node_update = pl.pallas_call(
    _node_body,
    grid=(1,),
    in_specs=[
        pl.BlockSpec((NC, N, D), lambda i: (0, 0, 0)),
        pl.BlockSpec((NC, DROWS * 16, 8), lambda i: (0, 0, 0)),
        pl.BlockSpec((N, D), lambda i: (0, 0)),
        pl.BlockSpec((1, D), lambda i: (0, 0)),
        pl.BlockSpec((1, D), lambda i: (0, 0)),
    ],
    out_specs=pl.BlockSpec((N, D), lambda i: (0, 0)),
    out_shape=jax.ShapeDtypeStruct((N, D), _f32),
)


def _eupd_body(ehat_ref, e_ref, stats_ref, gam_ref, bet_ref, out_ref):
    s1 = stats_ref[0:1, :]
    s2 = stats_ref[1:2, :]
    mu = s1 * (1.0 / E)
    var = s2 * (1.0 / E) - mu * mu
    ehat = ehat_ref[...]
    y = gam_ref[...] * (ehat - mu) / jnp.sqrt(var + 1e-5) + bet_ref[...]
    out_ref[...] = jnp.maximum(y, 0.0) + e_ref[...]


_edge_update = pl.pallas_call(
    _eupd_body,
    grid=(GE,),
    in_specs=[
        pl.BlockSpec((BE, D), lambda i: (i, 0)),
        pl.BlockSpec((BE, D), lambda i: (i, 0)),
        pl.BlockSpec((8, 128), lambda i: (0, 0)),
        pl.BlockSpec((1, D), lambda i: (0, 0)),
        pl.BlockSpec((1, D), lambda i: (0, 0)),
    ],
    out_specs=pl.BlockSpec((BE, D), lambda i: (i, 0)),
    out_shape=jax.ShapeDtypeStruct((E, D), _f32),
)


def kernel(h, e, edge_index, Wsrc, Wdst, We, Wv, attn,
           gamma_h, beta_h, gamma_e, beta_e):
    src = edge_index[0].astype(jnp.int32)
    dst = edge_index[1].astype(jnp.int32)
    for l in range(NL):
        p, q, v = _pqv(h, Wsrc[l], Wdst[l], Wv[l])
        g = _sc_gather_sum(p, q, src, dst)
        ehat, logit3, stats, lmax = _edge_stage(
            e, g, We[l], attn[l].reshape(1, D))
        gvec = jnp.full((LANES,), jnp.max(lmax), _f32)
        num, denp = _sc_aggregate(v, src, dst, logit3.reshape(E), gvec)
        den16 = denp.reshape(NC, DROWS * 16, 8)
        h = _node_update(num, den16, h,
                         gamma_h[l].reshape(1, D), beta_h[l].reshape(1, D))
        e = _edge_update(ehat, e, stats,
                         gamma_e[l].reshape(1, D), beta_e[l].reshape(1, D))
    return (h, e)


# speedup vs baseline: 1.1634x; 1.1066x over previous
import functools

import jax
import jax.numpy as jnp
from jax import lax
from jax.experimental import pallas as pl
from jax.experimental.pallas import tpu as pltpu
from jax.experimental.pallas import tpu_sc as plsc

N = 10000
E = 320000
D = 128
NL = 3

NC = 2
NS = 16
LANES = 16
NW = NC * NS
CE = E // NW
KCH = 80
NCH = CE // KCH

DROWS = 752
NACC = N + DROWS
ZSUB = NACC // NS
NZSUB = 10
RSUB = N // NZSUB

BE = 2560
GE = E // BE
BNODE = 2000
GNODE = N // BNODE

_mesh = plsc.VectorSubcoreMesh(
    core_axis_name="c", subcore_axis_name="s", num_cores=NC, num_subcores=NS)

_f32 = jnp.float32


def _sc_gather_sum_body(p_hbm, q_hbm, src_hbm, dst_hbm, g_hbm,
                        sslab, dslab, buf0, buf1, semp, sems):
    wid = lax.axis_index("s") * NC + lax.axis_index("c")
    base = wid * CE

    pltpu.sync_copy(src_hbm.at[pl.ds(base, CE)], sslab)
    pltpu.sync_copy(dst_hbm.at[pl.ds(base, CE)], dslab)

    def wait_one(buf, sm):
        pltpu.make_async_copy(g_hbm.at[pl.ds(base, KCH)], buf, sm).wait()

    def stage(i, b, nb, dr, pf):
        co = i * KCH
        wait_one(b, semp)
        if pf:
            if dr:
                wait_one(nb, sems)
            pltpu.async_copy(p_hbm.at[sslab.at[pl.ds(co + KCH, KCH)]],
                             nb, semp)
        pltpu.sync_copy(q_hbm.at[dslab.at[pl.ds(co, KCH)]], b, add=True)
        pltpu.async_copy(b, g_hbm.at[pl.ds(base + co, KCH)], sems)

    pltpu.async_copy(p_hbm.at[sslab.at[pl.ds(0, KCH)]], buf0, semp)
    stage(0, buf0, buf1, False, True)

    def chunk2(j, carry):
        i = j * 2 + 1
        stage(i, buf1, buf0, True, True)
        stage(i + 1, buf0, buf1, True, True)
        return carry

    lax.fori_loop(0, (NCH - 3) // 2, chunk2, 0, unroll=False)
    stage(NCH - 2, buf1, buf0, True, True)
    stage(NCH - 1, buf0, buf1, False, False)
    wait_one(buf0, sems)
    wait_one(buf1, sems)


_sc_gather_sum = functools.partial(
    pl.kernel,
    out_type=jax.ShapeDtypeStruct((E, D), _f32),
    mesh=_mesh,
    scratch_types=[
        pltpu.VMEM((CE,), jnp.int32),
        pltpu.VMEM((CE,), jnp.int32),
        pltpu.VMEM((KCH, D), _f32),
        pltpu.VMEM((KCH, D), _f32),
        pltpu.SemaphoreType.DMA,
        pltpu.SemaphoreType.DMA,
    ],
)(_sc_gather_sum_body)


def _sc_aggregate_body(v_hbm, src_hbm, dst_hbm, logit_hbm, gmax_hbm,
                       num_out, den_out,
                       sslab, lbuf, didx, didxp, didx8, exbuf,
                       rowbuf, rowbuf2, exrow, gbuf, acc_sh, semv):
    cid = lax.axis_index("c")
    sid = lax.axis_index("s")
    wid = sid * NC + cid

    pltpu.sync_copy(gmax_hbm, gbuf)
    base = wid * CE
    pltpu.sync_copy(src_hbm.at[pl.ds(base, CE)], sslab)
    gv = gbuf[...]

    zero = jnp.zeros((LANES,), _f32)

    def zrow2(k, c2):
        for j in range(D // LANES):
            exrow[k, pl.ds(j * LANES, LANES)] = zero
        return c2

    lax.fori_loop(0, KCH, zrow2, 0, unroll=False)

    for t in range(ZSUB // KCH):
        pltpu.sync_copy(exrow, acc_sh.at[pl.ds(sid * ZSUB + t * KCH, KCH)])
    pltpu.sync_copy(exrow.at[pl.ds(0, ZSUB % KCH)],
                    acc_sh.at[pl.ds(sid * ZSUB + ZSUB - ZSUB % KCH,
                                    ZSUB % KCH)])
    plsc.subcore_barrier()

    lane0 = lax.iota(jnp.int32, LANES)

    def wait_v(buf):
        pltpu.make_async_copy(v_hbm.at[pl.ds(0, KCH)], buf, semv).wait()

    def body(i, cur, nxt, pf):
        co = i * KCH
        pltpu.sync_copy(logit_hbm.at[pl.ds(base + co, KCH)], lbuf)
        pltpu.sync_copy(dst_hbm.at[pl.ds(base + co, KCH)], didx)
        for g in range(KCH // LANES):
            sl = pl.ds(g * LANES, LANES)
            dv = didx[sl]
            didxp[sl] = dv
            didx8[sl] = (dv >> 4) + N
            exbuf[sl] = jnp.exp(lbuf[sl] - gv)
        wait_v(cur)
        if pf:
            pltpu.async_copy(v_hbm.at[sslab.at[pl.ds(co + KCH, KCH)]],
                             nxt, semv)

        def row(k, c2):
            s = exbuf[pl.ds(k, LANES)][0]
            dk = didxp[pl.ds(k, LANES)][0]
            slot = dk & 15
            lane = (slot >> 1) * LANES
            pos = (slot & 1) * 8
            for j in range(D // LANES):
                sl2 = pl.ds(j * LANES, LANES)
                cur[k, sl2] = cur[k, sl2] * s
                exrow[k, sl2] = zero
            sv = jnp.where(lane0 == pos, s, 0.0)
            exrow[k, pl.ds(lane, LANES)] = sv
            return c2

        lax.fori_loop(0, KCH, row, 0, unroll=False)
        pltpu.sync_copy(cur, acc_sh.at[didx], add=True)
        pltpu.sync_copy(exrow, acc_sh.at[didx8], add=True)

    pltpu.async_copy(v_hbm.at[sslab.at[pl.ds(0, KCH)]], rowbuf, semv)

    def chunk2(j, carry):
        i = j * 2
        body(i, rowbuf, rowbuf2, True)
        body(i + 1, rowbuf2, rowbuf, True)
        return carry

    lax.fori_loop(0, (NCH - 1) // 2, chunk2, 0, unroll=False)
    body(NCH - 1, rowbuf, rowbuf2, False)

    plsc.subcore_barrier()

    @pl.when(sid < NZSUB)
    def _():
        pltpu.sync_copy(acc_sh.at[pl.ds(sid * RSUB, RSUB)],
                        num_out.at[cid, pl.ds(sid * RSUB, RSUB)])

    @pl.when(sid == NZSUB)
    def _():
        pltpu.sync_copy(acc_sh.at[pl.ds(N, DROWS)], den_out.at[cid])


_sc_aggregate = functools.partial(
    pl.kernel,
    out_type=(
        jax.ShapeDtypeStruct((NC, N, D), _f32),
        jax.ShapeDtypeStruct((NC, DROWS, D), _f32),
    ),
    mesh=_mesh,
    scratch_types=[
        pltpu.VMEM((CE,), jnp.int32),
        pltpu.VMEM((KCH,), _f32),
        pltpu.VMEM((KCH,), jnp.int32),
        pltpu.VMEM((KCH + LANES,), jnp.int32),
        pltpu.VMEM((KCH,), jnp.int32),
        pltpu.VMEM((KCH + LANES,), _f32),
        pltpu.VMEM((KCH, D), _f32),
        pltpu.VMEM((KCH, D), _f32),
        pltpu.VMEM((KCH, D), _f32),
        pltpu.VMEM((LANES,), _f32),
        pltpu.VMEM_SHARED((NACC, D), _f32),
        pltpu.SemaphoreType.DMA,
    ],
)(_sc_aggregate_body)


def _pqv_body(h_ref, ws_ref, wd_ref, wv_ref, p_ref, q_ref, v_ref):
    hb = h_ref[...]
    p_ref[...] = jnp.dot(hb, ws_ref[...], preferred_element_type=_f32)
    q_ref[...] = jnp.dot(hb, wd_ref[...], preferred_element_type=_f32)
    v_ref[...] = jnp.dot(hb, wv_ref[...], preferred_element_type=_f32)


_pqv = pl.pallas_call(
    _pqv_body,
    grid=(GNODE,),
    in_specs=[
        pl.BlockSpec((BNODE, D), lambda i: (i, 0)),
        pl.BlockSpec((D, D), lambda i: (0, 0)),
        pl.BlockSpec((D, D), lambda i: (0, 0)),
        pl.BlockSpec((D, D), lambda i: (0, 0)),
    ],
    out_specs=[
        pl.BlockSpec((BNODE, D), lambda i: (i, 0)),
        pl.BlockSpec((BNODE, D), lambda i: (i, 0)),
        pl.BlockSpec((BNODE, D), lambda i: (i, 0)),
    ],
    out_shape=[jax.ShapeDtypeStruct((N, D), _f32)] * 3,
)


def _edge_body(e_ref, g_ref, we_ref, attn_ref,
               ehat_ref, logit_ref, stats_ref, lmax_ref):
    i = pl.program_id(0)
    ehat = g_ref[...] + jnp.dot(e_ref[...], we_ref[...],
                                preferred_element_type=_f32)
    ehat_ref[...] = ehat
    lr = jnp.where(ehat > 0, ehat, 0.2 * ehat)
    logit_row = lax.dot_general(attn_ref[...], lr, (((1,), (1,)), ((), ())),
                                preferred_element_type=_f32)
    logit_ref[...] = logit_row.reshape(1, 1, BE)

    @pl.when(i == 0)
    def _():
        stats_ref[...] = jnp.zeros_like(stats_ref)
        lmax_ref[...] = jnp.full_like(lmax_ref, -jnp.inf)

    stats_ref[0:1, :] += jnp.sum(ehat, axis=0, keepdims=True)
    stats_ref[1:2, :] += jnp.sum(ehat * ehat, axis=0, keepdims=True)
    lmax_ref[...] = jnp.maximum(lmax_ref[...], jnp.max(logit_row))


_edge_stage = pl.pallas_call(
    _edge_body,
    grid=(GE,),
    in_specs=[
        pl.BlockSpec((BE, D), lambda i: (i, 0)),
        pl.BlockSpec((BE, D), lambda i: (i, 0)),
        pl.BlockSpec((D, D), lambda i: (0, 0)),
        pl.BlockSpec((1, D), lambda i: (0, 0)),
    ],
    out_specs=[
        pl.BlockSpec((BE, D), lambda i: (i, 0)),
        pl.BlockSpec((1, 1, BE), lambda i: (i, 0, 0)),
        pl.BlockSpec((8, 128), lambda i: (0, 0)),
        pl.BlockSpec((8, 128), lambda i: (0, 0)),
    ],
    out_shape=[
        jax.ShapeDtypeStruct((E, D), _f32),
        jax.ShapeDtypeStruct((GE, 1, BE), _f32),
        jax.ShapeDtypeStruct((8, 128), _f32),
        jax.ShapeDtypeStruct((8, 128), _f32),
    ],
)


def _node_body(num_ref, den_ref, h_ref, gam_ref, bet_ref, out_ref):
    num = num_ref[0] + num_ref[1]
    den = den_ref[0, :N, :] + den_ref[1, :N, :]
    agg = num / (den[:, 0:1] + 1e-16)
    mu = jnp.mean(agg, axis=0, keepdims=True)
    var = jnp.mean(agg * agg, axis=0, keepdims=True) - mu * mu
    y = gam_ref[...] * (agg - mu) / jnp.sqrt(var + 1e-5) + bet_ref[...]
    out_ref[...] = jnp.maximum(y, 0.0) + h_ref[...]


_node_update = pl.pallas_call(
    _node_body,
    grid=(1,),
    in_specs=[
        pl.BlockSpec((NC, N, D), lambda i: (0, 0, 0)),
        pl.BlockSpec((NC, DROWS * 16, 8), lambda i: (0, 0, 0)),
        pl.BlockSpec((N, D), lambda i: (0, 0)),
        pl.BlockSpec((1, D), lambda i: (0, 0)),
        pl.BlockSpec((1, D), lambda i: (0, 0)),
    ],
    out_specs=pl.BlockSpec((N, D), lambda i: (0, 0)),
    out_shape=jax.ShapeDtypeStruct((N, D), _f32),
)


def _eupd_body(ehat_ref, e_ref, stats_ref, gam_ref, bet_ref, out_ref):
    s1 = stats_ref[0:1, :]
    s2 = stats_ref[1:2, :]
    mu = s1 * (1.0 / E)
    var = s2 * (1.0 / E) - mu * mu
    ehat = ehat_ref[...]
    y = gam_ref[...] * (ehat - mu) / jnp.sqrt(var + 1e-5) + bet_ref[...]
    out_ref[...] = jnp.maximum(y, 0.0) + e_ref[...]


_edge_update = pl.pallas_call(
    _eupd_body,
    grid=(GE,),
    in_specs=[
        pl.BlockSpec((BE, D), lambda i: (i, 0)),
        pl.BlockSpec((BE, D), lambda i: (i, 0)),
        pl.BlockSpec((8, 128), lambda i: (0, 0)),
        pl.BlockSpec((1, D), lambda i: (0, 0)),
        pl.BlockSpec((1, D), lambda i: (0, 0)),
    ],
    out_specs=pl.BlockSpec((BE, D), lambda i: (i, 0)),
    out_shape=jax.ShapeDtypeStruct((E, D), _f32),
)


def kernel(h, e, edge_index, Wsrc, Wdst, We, Wv, attn,
           gamma_h, beta_h, gamma_e, beta_e):
    src = edge_index[0].astype(jnp.int32)
    dst = edge_index[1].astype(jnp.int32)
    for l in range(NL):
        p, q, v = _pqv(h, Wsrc[l], Wdst[l], Wv[l])
        g = _sc_gather_sum(p, q, src, dst)
        ehat, logit3, stats, lmax = _edge_stage(
            e, g, We[l], attn[l].reshape(1, D))
        gvec = jnp.full((LANES,), jnp.max(lmax), _f32)
        num, denp = _sc_aggregate(v, src, dst, logit3.reshape(E), gvec)
        den16 = denp.reshape(NC, DROWS * 16, 8)
        h = _node_update(num, den16, h,
                         gamma_h[l].reshape(1, D), beta_h[l].reshape(1, D))
        e = _edge_update(ehat, e, stats,
                         gamma_e[l].reshape(1, D), beta_e[l].reshape(1, D))
    return (h, e)
